# Initial kernel scaffold; baseline (speedup 1.0000x reference)
import jax
import jax.numpy as jnp
from jax.experimental import pallas as pl


def kernel(X):
    raise NotImplementedError("write your pallas kernel here")


# speedup vs baseline: 3.6081x; 3.6081x over previous
import jax
import jax.numpy as jnp
from jax.experimental import pallas as pl
from jax.experimental.pallas import tpu as pltpu

N = 10000
D = 128
TM = 512
TN = 512
NPAD = 10240
NI = NPAD // TM
NJ = NPAD // TN

_EPS = 1e-10
_REG = 2.0 / 3.0


def _norm_kernel(x_ref, f_ref):
    x = x_ref[...]
    nrm = jnp.sqrt(jnp.sum(x * x, axis=1, keepdims=True))
    f_ref[...] = x / jnp.maximum(nrm, 1e-12)


def _masked_sim(f_i, f_j, i, j):
    s = jax.lax.dot_general(
        f_i, f_j, (((1,), (1,)), ((), ())),
        preferred_element_type=jnp.float32)
    rows = i * TM + jax.lax.broadcasted_iota(jnp.int32, s.shape, 0)
    cols = j * TN + jax.lax.broadcasted_iota(jnp.int32, s.shape, 1)
    return jnp.where((s >= _EPS) & (rows != cols), s, 0.0)


def _rowsum_kernel(f_i_ref, f_j_ref, d_ref, acc_ref):
    i = pl.program_id(0)
    j = pl.program_id(1)

    @pl.when(j == 0)
    def _():
        acc_ref[...] = jnp.zeros_like(acc_ref)

    s = _masked_sim(f_i_ref[...], f_j_ref[...], i, j)
    ps = s[:, 0:128]
    for k in range(1, TN // 128):
        ps = ps + s[:, k * 128:(k + 1) * 128]
    acc_ref[...] += ps

    @pl.when(j == NJ - 1)
    def _():
        rowsum = jnp.sum(acc_ref[...], axis=1, keepdims=True) + 1.0
        d_ref[...] = jnp.broadcast_to(jax.lax.rsqrt(rowsum), d_ref.shape)


def _diffuse_kernel(x_i_ref, x_j_ref, f_i_ref, f_j_ref, d_i_ref, d_j_ref,
                    o_ref, acc_ref):
    i = pl.program_id(0)
    j = pl.program_id(1)

    @pl.when(j == 0)
    def _():
        acc_ref[...] = jnp.zeros_like(acc_ref)

    s = _masked_sim(f_i_ref[...], f_j_ref[...], i, j)
    y_j = d_j_ref[...] * x_j_ref[...]
    acc_ref[...] += jax.lax.dot_general(
        s, y_j, (((1,), (0,)), ((), ())),
        preferred_element_type=jnp.float32)

    @pl.when(j == NJ - 1)
    def _():
        x_i = x_i_ref[...]
        d_i = d_i_ref[...]
        y_i = d_i * x_i
        o_ref[...] = (1.0 - _REG) * x_i + _REG * d_i * (acc_ref[...] + y_i)


def _spec_i(bs):
    return pl.BlockSpec(bs, lambda i, j: (i, 0))


def _spec_j(bs):
    return pl.BlockSpec(bs, lambda i, j: (j, 0))


@jax.jit
def kernel(X):
    Xp = jnp.pad(X, ((0, NPAD - N), (0, 0)))

    F = pl.pallas_call(
        _norm_kernel,
        grid=(NI,),
        in_specs=[pl.BlockSpec((TM, D), lambda i: (i, 0))],
        out_specs=pl.BlockSpec((TM, D), lambda i: (i, 0)),
        out_shape=jax.ShapeDtypeStruct((NPAD, D), jnp.float32),
    )(Xp)

    DB = pl.pallas_call(
        _rowsum_kernel,
        grid=(NI, NJ),
        in_specs=[_spec_i((TM, D)), _spec_j((TN, D))],
        out_specs=_spec_i((TM, D)),
        out_shape=jax.ShapeDtypeStruct((NPAD, D), jnp.float32),
        scratch_shapes=[pltpu.VMEM((TM, D), jnp.float32)],
        compiler_params=pltpu.CompilerParams(
            dimension_semantics=("arbitrary", "arbitrary")),
    )(F, F)

    OUT = pl.pallas_call(
        _diffuse_kernel,
        grid=(NI, NJ),
        in_specs=[_spec_i((TM, D)), _spec_j((TN, D)),
                  _spec_i((TM, D)), _spec_j((TN, D)),
                  _spec_i((TM, D)), _spec_j((TN, D))],
        out_specs=_spec_i((TM, D)),
        out_shape=jax.ShapeDtypeStruct((NPAD, D), jnp.float32),
        scratch_shapes=[pltpu.VMEM((TM, D), jnp.float32)],
        compiler_params=pltpu.CompilerParams(
            dimension_semantics=("arbitrary", "arbitrary")),
    )(Xp, Xp, F, F, DB, DB)

    return OUT[:N]
